# baseline (speedup 1.0000x reference)
import functools

import jax
import jax.numpy as jnp
from jax.experimental import pallas as pl
from jax.experimental.pallas import tpu as pltpu

_OUT = 7
_SCALE = 1.0 / 16.0
_NEG = float("-inf")


def _pool_kernel(ws_r, we_r, hs_r, he_r, ws4_r, hs4_r, ws9_r, hs9_r,
                 wide_r, bidx_r, x_r, out_r, cm_r, *, rb, H, C):
    i = pl.program_id(0)

    def width_stage(r, b, starts_r, nw):
        for pw in range(_OUT):
            st = starts_r[r * _OUT + pw]
            lo = ws_r[r * _OUT + pw]
            hi = we_r[r * _OUT + pw]
            sl = x_r[b, pl.ds(st, nw)]
            wi = jax.lax.broadcasted_iota(jnp.int32, (nw, 1, 1), 0) + st
            m = (wi >= lo) & (wi < hi)
            colmax = jnp.max(jnp.where(m, sl, _NEG), axis=0)
            cm_r[0, pw, 0:H, :] = colmax
            cm_r[1, pw, 0:H - 4, :] = colmax[4:]
            cm_r[2, pw, 0:H - 8, :] = colmax[8:]
            cm_r[3, pw, 0:H - 12, :] = colmax[12:]

    def height_stage(j, r, starts_r, copies, rows):
        gran_shift = 2 if copies == 4 else 3
        for ph in range(_OUT):
            st = starts_r[r * _OUT + ph]
            lo = hs_r[r * _OUT + ph]
            hi = he_r[r * _OUT + ph]
            base = pl.multiple_of((st >> 4) << 4, 16)
            sh = (st >> gran_shift) & (copies - 1)
            csel = sh if copies == 4 else 2 * sh
            ch = cm_r[csel, :, pl.ds(base, rows), :]
            hi_iota = (jax.lax.broadcasted_iota(jnp.int32, (1, rows, 1), 1)
                       + base + (sh << gran_shift))
            m = (hi_iota >= lo) & (hi_iota < hi)
            red = jnp.max(jnp.where(m, ch, _NEG), axis=1)
            res = jnp.where(red == _NEG, jnp.bfloat16(0.0), red)
            out_r[j, ph * _OUT:(ph + 1) * _OUT, :] = res[:_OUT]

    for j in range(rb):
        r = i * rb + j
        b = bidx_r[r]
        wide = wide_r[r]

        @pl.when(wide == 0)
        def _():
            width_stage(r, b, ws4_r, 4)
            height_stage(j, r, hs4_r, 4, 8)

        @pl.when(wide != 0)
        def _():
            width_stage(r, b, ws9_r, 9)
            height_stage(j, r, hs9_r, 2, 16)


def _mlp_kernel(flat_r, w1_r, b1_r, w2_r, b2_r, wl_r, bl_r, wsc_r, bsc_r,
                locs_r, scores_r, acc_r, *, kg):
    k = pl.program_id(1)

    @pl.when(k == 0)
    def _():
        acc_r[...] = jnp.zeros_like(acc_r)

    bf = jnp.bfloat16
    dn = (((1,), (1,)), ((), ()))
    acc_r[...] += jax.lax.dot_general(
        flat_r[...], w1_r[...].astype(bf), dn,
        preferred_element_type=jnp.float32)

    @pl.when(k == kg - 1)
    def _():
        h1 = jnp.maximum(acc_r[...] + b1_r[...], 0.0).astype(bf)
        h2 = jnp.maximum(
            jnp.dot(h1, w2_r[...], preferred_element_type=jnp.float32)
            + b2_r[...], 0.0).astype(bf)
        locs_r[...] = jnp.dot(
            h2, wl_r[...], preferred_element_type=jnp.float32) + bl_r[...]
        scores_r[...] = jnp.dot(
            h2, wsc_r[...], preferred_element_type=jnp.float32) + bsc_r[...]


def kernel(x, rois, roi_indices, w1, b1, w2, b2, w_loc, b_loc, w_score, b_score):
    B, C, H, W = x.shape
    R = rois.shape[0]
    fc = w1.shape[0]
    nl = w_loc.shape[0]
    ns = w_score.shape[0]
    bf = jnp.bfloat16

    xyxy = rois[:, jnp.array([1, 0, 3, 2])]
    s = jnp.round(xyxy * _SCALE).astype(jnp.int32)
    x1, y1, x2, y2 = s[:, 0], s[:, 1], s[:, 2], s[:, 3]
    bw = jnp.maximum(x2 - x1, 1).astype(jnp.float32) / _OUT
    bh = jnp.maximum(y2 - y1, 1).astype(jnp.float32) / _OUT
    p = jnp.arange(_OUT, dtype=jnp.float32)
    hs = jnp.clip(jnp.floor(p[None, :] * bh[:, None]).astype(jnp.int32)
                  + y1[:, None], 0, H)
    he = jnp.clip(jnp.ceil((p[None, :] + 1.0) * bh[:, None]).astype(jnp.int32)
                  + y1[:, None], 0, H)
    ws = jnp.clip(jnp.floor(p[None, :] * bw[:, None]).astype(jnp.int32)
                  + x1[:, None], 0, W)
    we = jnp.clip(jnp.ceil((p[None, :] + 1.0) * bw[:, None]).astype(jnp.int32)
                  + x1[:, None], 0, W)
    ws4 = jnp.minimum(ws, W - 4)
    hs4 = jnp.minimum(hs, H - 4)
    ws9 = jnp.minimum(ws, W - 9)
    hs9 = jnp.minimum(hs, H - 9)
    wide = (jnp.maximum(jnp.max(we - ws, axis=1),
                        jnp.max(he - hs, axis=1)) > 4).astype(jnp.int32)
    flat1 = lambda a: a.reshape(-1)

    x_t = jnp.transpose(x, (0, 3, 2, 1)).astype(bf)
    w2t = jnp.transpose(w2).astype(bf)
    wlt = jnp.transpose(w_loc).astype(bf)
    wst = jnp.transpose(w_score).astype(bf)

    rb = 16
    smem = pl.BlockSpec(memory_space=pltpu.SMEM)
    pool = pl.pallas_call(
        functools.partial(_pool_kernel, rb=rb, H=H, C=C),
        grid=(R // rb,),
        in_specs=[smem] * 10 + [
            pl.BlockSpec((B, W, H, C), lambda i: (0, 0, 0, 0)),
        ],
        out_specs=pl.BlockSpec((rb, _OUT * _OUT, C), lambda i: (i, 0, 0)),
        out_shape=jax.ShapeDtypeStruct((R, _OUT * _OUT, C), bf),
        scratch_shapes=[pltpu.VMEM((4, 8, 64, C), bf)],
        compiler_params=pltpu.CompilerParams(
            dimension_semantics=("arbitrary",)),
        name="roi_max_pool",
    )(flat1(ws), flat1(we), flat1(hs), flat1(he), flat1(ws4), flat1(hs4),
      flat1(ws9), flat1(hs9), wide, roi_indices, x_t)

    flat = jax.lax.reshape(pool, (R, C * _OUT * _OUT), dimensions=(0, 2, 1))

    rg = 2
    kg = 7
    kb = (C * _OUT * _OUT) // kg
    mr = R // rg
    locs, scores = pl.pallas_call(
        functools.partial(_mlp_kernel, kg=kg),
        grid=(rg, kg),
        in_specs=[
            pl.BlockSpec((mr, kb), lambda i, k: (i, k)),
            pl.BlockSpec((fc, kb), lambda i, k: (0, k)),
            pl.BlockSpec((1, fc), lambda i, k: (0, 0)),
            pl.BlockSpec((fc, fc), lambda i, k: (0, 0)),
            pl.BlockSpec((1, fc), lambda i, k: (0, 0)),
            pl.BlockSpec((fc, nl), lambda i, k: (0, 0)),
            pl.BlockSpec((1, nl), lambda i, k: (0, 0)),
            pl.BlockSpec((fc, ns), lambda i, k: (0, 0)),
            pl.BlockSpec((1, ns), lambda i, k: (0, 0)),
        ],
        out_specs=[
            pl.BlockSpec((mr, nl), lambda i, k: (i, 0)),
            pl.BlockSpec((mr, ns), lambda i, k: (i, 0)),
        ],
        out_shape=[
            jax.ShapeDtypeStruct((R, nl), jnp.float32),
            jax.ShapeDtypeStruct((R, ns), jnp.float32),
        ],
        scratch_shapes=[pltpu.VMEM((mr, fc), jnp.float32)],
        compiler_params=pltpu.CompilerParams(
            dimension_semantics=("parallel", "arbitrary")),
        name="roi_mlp_head",
    )(flat, w1, b1.reshape(1, fc), w2t, b2.reshape(1, fc),
      wlt, b_loc.reshape(1, nl), wst, b_score.reshape(1, ns))

    return (locs, scores)


# speedup vs baseline: 1.0452x; 1.0452x over previous
import functools

import jax
import jax.numpy as jnp
from jax.experimental import pallas as pl
from jax.experimental.pallas import tpu as pltpu

_OUT = 7
_SCALE = 1.0 / 16.0
_NEG = float("-inf")


def _pool_kernel(ws_r, we_r, hs_r, he_r, ws4_r, hs4_r, ws9_r, hs9_r,
                 wide_r, bidx_r, x_r, out_r, cm_r, *, rb, H, C):
    i = pl.program_id(0)

    def width_stage(r, b, starts_r, nw):
        for pw in range(_OUT):
            st = starts_r[r * _OUT + pw]
            lo = ws_r[r * _OUT + pw]
            hi = we_r[r * _OUT + pw]
            sl = x_r[b, pl.ds(st, nw)]
            wi = jax.lax.broadcasted_iota(jnp.int32, (nw, 1, 1), 0) + st
            m = (wi >= lo) & (wi < hi)
            colmax = jnp.max(jnp.where(m, sl, _NEG), axis=0)
            cm_r[0, pw, 0:H, :] = colmax
            cm_r[1, pw, 0:H - 4, :] = colmax[4:]
            cm_r[2, pw, 0:H - 8, :] = colmax[8:]
            cm_r[3, pw, 0:H - 12, :] = colmax[12:]

    def height_stage(j, r, starts_r, copies, rows):
        gran_shift = 2 if copies == 4 else 3
        for ph in range(_OUT):
            st = starts_r[r * _OUT + ph]
            lo = hs_r[r * _OUT + ph]
            hi = he_r[r * _OUT + ph]
            base = pl.multiple_of((st >> 4) << 4, 16)
            sh = (st >> gran_shift) & (copies - 1)
            csel = sh if copies == 4 else 2 * sh
            ch = cm_r[csel, :, pl.ds(base, rows), :]
            hi_iota = (jax.lax.broadcasted_iota(jnp.int32, (1, rows, 1), 1)
                       + base + (sh << gran_shift))
            m = (hi_iota >= lo) & (hi_iota < hi)
            red = jnp.max(jnp.where(m, ch, _NEG), axis=1)
            res = jnp.where(red == _NEG, jnp.bfloat16(0.0), red)
            out_r[j, ph * _OUT:(ph + 1) * _OUT, :] = res[:_OUT]

    for j in range(rb):
        r = i * rb + j
        b = bidx_r[r]
        wide = wide_r[r]

        @pl.when(wide == 0)
        def _():
            width_stage(r, b, ws4_r, 4)
            height_stage(j, r, hs4_r, 4, 8)

        @pl.when(wide != 0)
        def _():
            width_stage(r, b, ws9_r, 9)
            height_stage(j, r, hs9_r, 2, 16)


def _mlp_kernel(flat_r, w1_r, b1_r, w2_r, b2_r, wl_r, bl_r, wsc_r, bsc_r,
                locs_r, scores_r, acc_r, *, kg):
    k = pl.program_id(1)

    @pl.when(k == 0)
    def _():
        acc_r[...] = jnp.zeros_like(acc_r)

    bf = jnp.bfloat16
    dn = (((1,), (1,)), ((), ()))
    acc_r[...] += jax.lax.dot_general(
        flat_r[...], w1_r[...].astype(bf), dn,
        preferred_element_type=jnp.float32)

    @pl.when(k == kg - 1)
    def _():
        h1 = jnp.maximum(acc_r[...] + b1_r[...], 0.0).astype(bf)
        h2 = jnp.maximum(
            jnp.dot(h1, w2_r[...], preferred_element_type=jnp.float32)
            + b2_r[...], 0.0).astype(bf)
        locs_r[...] = jnp.dot(
            h2, wl_r[...], preferred_element_type=jnp.float32) + bl_r[...]
        scores_r[...] = jnp.dot(
            h2, wsc_r[...], preferred_element_type=jnp.float32) + bsc_r[...]


def kernel(x, rois, roi_indices, w1, b1, w2, b2, w_loc, b_loc, w_score, b_score):
    B, C, H, W = x.shape
    R = rois.shape[0]
    fc = w1.shape[0]
    nl = w_loc.shape[0]
    ns = w_score.shape[0]
    bf = jnp.bfloat16

    xyxy = rois[:, jnp.array([1, 0, 3, 2])]
    s = jnp.round(xyxy * _SCALE).astype(jnp.int32)
    x1, y1, x2, y2 = s[:, 0], s[:, 1], s[:, 2], s[:, 3]
    bw = jnp.maximum(x2 - x1, 1).astype(jnp.float32) / _OUT
    bh = jnp.maximum(y2 - y1, 1).astype(jnp.float32) / _OUT
    p = jnp.arange(_OUT, dtype=jnp.float32)
    hs = jnp.clip(jnp.floor(p[None, :] * bh[:, None]).astype(jnp.int32)
                  + y1[:, None], 0, H)
    he = jnp.clip(jnp.ceil((p[None, :] + 1.0) * bh[:, None]).astype(jnp.int32)
                  + y1[:, None], 0, H)
    ws = jnp.clip(jnp.floor(p[None, :] * bw[:, None]).astype(jnp.int32)
                  + x1[:, None], 0, W)
    we = jnp.clip(jnp.ceil((p[None, :] + 1.0) * bw[:, None]).astype(jnp.int32)
                  + x1[:, None], 0, W)
    ws4 = jnp.minimum(ws, W - 4)
    hs4 = jnp.minimum(hs, H - 4)
    ws9 = jnp.minimum(ws, W - 9)
    hs9 = jnp.minimum(hs, H - 9)
    wide = (jnp.maximum(jnp.max(we - ws, axis=1),
                        jnp.max(he - hs, axis=1)) > 4).astype(jnp.int32)
    flat1 = lambda a: a.reshape(-1)

    x_t = jnp.transpose(x, (0, 3, 2, 1)).astype(bf)
    w2t = jnp.transpose(w2).astype(bf)
    wlt = jnp.transpose(w_loc).astype(bf)
    wst = jnp.transpose(w_score).astype(bf)

    rb = 16
    smem = pl.BlockSpec(memory_space=pltpu.SMEM)
    pool = pl.pallas_call(
        functools.partial(_pool_kernel, rb=rb, H=H, C=C),
        grid=(R // rb,),
        in_specs=[smem] * 10 + [
            pl.BlockSpec((B, W, H, C), lambda i: (0, 0, 0, 0)),
        ],
        out_specs=pl.BlockSpec((rb, _OUT * _OUT, C), lambda i: (i, 0, 0)),
        out_shape=jax.ShapeDtypeStruct((R, _OUT * _OUT, C), bf),
        scratch_shapes=[pltpu.VMEM((4, 8, 64, C), bf)],
        compiler_params=pltpu.CompilerParams(
            dimension_semantics=("arbitrary",)),
        name="roi_max_pool",
    )(flat1(ws), flat1(we), flat1(hs), flat1(he), flat1(ws4), flat1(hs4),
      flat1(ws9), flat1(hs9), wide, roi_indices, x_t)

    flat = jax.lax.reshape(pool, (R, C * _OUT * _OUT), dimensions=(0, 2, 1))

    rg = 1
    kg = 7
    kb = (C * _OUT * _OUT) // kg
    mr = R // rg
    locs, scores = pl.pallas_call(
        functools.partial(_mlp_kernel, kg=kg),
        grid=(rg, kg),
        in_specs=[
            pl.BlockSpec((mr, kb), lambda i, k: (i, k)),
            pl.BlockSpec((fc, kb), lambda i, k: (0, k)),
            pl.BlockSpec((1, fc), lambda i, k: (0, 0)),
            pl.BlockSpec((fc, fc), lambda i, k: (0, 0)),
            pl.BlockSpec((1, fc), lambda i, k: (0, 0)),
            pl.BlockSpec((fc, nl), lambda i, k: (0, 0)),
            pl.BlockSpec((1, nl), lambda i, k: (0, 0)),
            pl.BlockSpec((fc, ns), lambda i, k: (0, 0)),
            pl.BlockSpec((1, ns), lambda i, k: (0, 0)),
        ],
        out_specs=[
            pl.BlockSpec((mr, nl), lambda i, k: (i, 0)),
            pl.BlockSpec((mr, ns), lambda i, k: (i, 0)),
        ],
        out_shape=[
            jax.ShapeDtypeStruct((R, nl), jnp.float32),
            jax.ShapeDtypeStruct((R, ns), jnp.float32),
        ],
        scratch_shapes=[pltpu.VMEM((mr, fc), jnp.float32)],
        compiler_params=pltpu.CompilerParams(
            dimension_semantics=("parallel", "arbitrary")),
        name="roi_mlp_head",
    )(flat, w1, b1.reshape(1, fc), w2t, b2.reshape(1, fc),
      wlt, b_loc.reshape(1, nl), wst, b_score.reshape(1, ns))

    return (locs, scores)
